# baseline (speedup 1.0000x reference)
import functools

import jax
import jax.numpy as jnp
from jax import lax
from jax.experimental import pallas as pl
from jax.experimental.pallas import tpu as pltpu
from jax.experimental.pallas import tpu_sc as plsc

N = 256
CUTOFF = 10
LN2 = 0.6931471805599453
L = 16

_SC_MESH = plsc.VectorSubcoreMesh(core_axis_name="c", subcore_axis_name="s")


@functools.partial(
    pl.kernel,
    mesh=_SC_MESH,
    out_type=jax.ShapeDtypeStruct((2, N), jnp.int32),
    scratch_types=[
        pltpu.VMEM((2 * (N + L),), jnp.float32),
        pltpu.VMEM((L,), jnp.int32),
    ],
)
def _sc_ranks(o_hbm, s_hbm, ranks_hbm, vals_v, rank_v):
    wid = lax.axis_index("s") * 2 + lax.axis_index("c")
    half = wid // 16
    half_base = half * (N + L)
    local = (wid % 16) * L
    pltpu.sync_copy(o_hbm, vals_v.at[pl.ds(0, N)])
    pltpu.sync_copy(o_hbm.at[pl.ds(0, L)], vals_v.at[pl.ds(N, L)])
    pltpu.sync_copy(s_hbm, vals_v.at[pl.ds(N + L, N)])
    pltpu.sync_copy(s_hbm.at[pl.ds(0, L)], vals_v.at[pl.ds(2 * N + L, L)])

    iota = lax.iota(jnp.int32, L)
    mine_idx = local + iota
    mine = vals_v[pl.ds(half_base + local, L)]

    def body(t, acc):
        v = vals_v[pl.ds(half_base + t, L)]
        kg = t + iota
        kidx = jnp.where(kg >= N, kg - N, kg)
        gt = jnp.where(v > mine, 1, 0)
        tie = jnp.where((v == mine) & (kidx < mine_idx), 1, 0)
        return acc + gt + tie

    acc = lax.fori_loop(0, N, body, jnp.zeros((L,), jnp.int32))
    rank_v[...] = acc
    pltpu.sync_copy(rank_v, ranks_hbm.at[half, pl.ds(local, L)])


def _combine_kernel(o_row, s_row, ranks2, out_ref):
    orow = o_row[...]
    srow = s_row[...]
    r = ranks2[...]

    def disc(rank_i32):
        rank = rank_i32.astype(jnp.float32)
        return jnp.where(rank < CUTOFF, LN2 / jnp.log(rank + 2.0), 0.0)

    d_row = disc(r[0:1, :])
    ds_row = disc(r[1:2, :])
    g_row = jnp.exp(srow * LN2) - 1.0
    idcg = jnp.sum(g_row * ds_row, axis=(0, 1), keepdims=True)

    ones = jnp.ones((1, N), jnp.float32)

    def colmat(x_row):
        return lax.dot_general(x_row, ones, (((0,), (0,)), ((), ())),
                               precision=lax.Precision.HIGHEST,
                               preferred_element_type=jnp.float32)

    oc = colmat(orow)
    gc = colmat(g_row)
    dc = colmat(d_row)

    diff = oc - orow
    logits = jax.nn.sigmoid(diff)
    log_p = jnp.maximum(jnp.log(logits), -100.0)
    log_1mp = jnp.maximum(jnp.log(1.0 - logits), -100.0)
    labels = jnp.where(gc > g_row, 1.0, 0.0)
    bce = -(labels * log_p + (1.0 - labels) * log_1mp)

    w = (jnp.abs(gc - g_row) * jnp.abs(dc - d_row)
         * jnp.where(oc != orow, 1.0, 0.0))
    total = jnp.sum(bce * w, axis=(0, 1), keepdims=True)
    out_ref[...] = total / (idcg * N)


def kernel(outputs, scores):
    o = outputs.reshape(-1)
    s = scores.reshape(-1)
    ranks = _sc_ranks(o, s)
    loss = pl.pallas_call(
        _combine_kernel,
        out_shape=jax.ShapeDtypeStruct((1, 1), jnp.float32),
    )(o.reshape(1, N), s.reshape(1, N), ranks)
    return loss.reshape(())


# speedup vs baseline: 1.0952x; 1.0952x over previous
import functools

import jax
import jax.numpy as jnp
from jax import lax
from jax.experimental import pallas as pl
from jax.experimental.pallas import tpu as pltpu
from jax.experimental.pallas import tpu_sc as plsc

N = 256
CUTOFF = 10
LN2 = 0.6931471805599453
L = 16

_SC_MESH = plsc.VectorSubcoreMesh(core_axis_name="c", subcore_axis_name="s")


@functools.partial(
    pl.kernel,
    mesh=_SC_MESH,
    out_type=jax.ShapeDtypeStruct((2, N), jnp.int32),
    scratch_types=[
        pltpu.VMEM((N + L,), jnp.float32),
        pltpu.VMEM((L,), jnp.int32),
    ],
)
def _sc_ranks(vals_hbm, ranks_hbm, half_v, rank_v):
    wid = lax.axis_index("s") * 2 + lax.axis_index("c")
    half = wid // 16
    local = (wid % 16) * L
    pltpu.sync_copy(vals_hbm.at[pl.ds(half * N, N)], half_v.at[pl.ds(0, N)])
    pltpu.sync_copy(vals_hbm.at[pl.ds(half * N, L)], half_v.at[pl.ds(N, L)])

    iota = lax.iota(jnp.int32, L)
    mine_idx = local + iota
    mine = half_v[pl.ds(local, L)]

    def body(t, acc):
        v = half_v[pl.ds(t, L)]
        kg = t + iota
        kidx = jnp.where(kg >= N, kg - N, kg)
        gt = jnp.where(v > mine, 1, 0)
        tie = jnp.where((v == mine) & (kidx < mine_idx), 1, 0)
        return acc + gt + tie

    acc = lax.fori_loop(0, N, body, jnp.zeros((L,), jnp.int32))
    rank_v[...] = acc
    pltpu.sync_copy(rank_v, ranks_hbm.at[half, pl.ds(local, L)])


def _combine_kernel(o_row, s_row, ranks2, out_ref):
    orow = o_row[...]
    srow = s_row[...]
    r = ranks2[...]

    def disc(rank_i32):
        rank = rank_i32.astype(jnp.float32)
        return jnp.where(rank < CUTOFF, LN2 / jnp.log(rank + 2.0), 0.0)

    d_row = disc(r[0:1, :])
    ds_row = disc(r[1:2, :])
    g_row = jnp.exp(srow * LN2) - 1.0
    idcg = jnp.sum(g_row * ds_row, axis=(0, 1), keepdims=True)

    ones = jnp.ones((1, N), jnp.float32)

    def colmat(x_row):
        return lax.dot_general(x_row, ones, (((0,), (0,)), ((), ())),
                               precision=lax.Precision.HIGHEST,
                               preferred_element_type=jnp.float32)

    oc = colmat(orow)
    gc = colmat(g_row)
    dc = colmat(d_row)

    diff = oc - orow
    logits = jax.nn.sigmoid(diff)
    log_p = jnp.maximum(jnp.log(logits), -100.0)
    log_1mp = jnp.maximum(jnp.log(1.0 - logits), -100.0)
    labels = jnp.where(gc > g_row, 1.0, 0.0)
    bce = -(labels * log_p + (1.0 - labels) * log_1mp)

    w = (jnp.abs(gc - g_row) * jnp.abs(dc - d_row)
         * jnp.where(oc != orow, 1.0, 0.0))
    total = jnp.sum(bce * w, axis=(0, 1), keepdims=True)
    out_ref[...] = total / (idcg * N)


def kernel(outputs, scores):
    o = outputs.reshape(-1)
    s = scores.reshape(-1)
    ranks = _sc_ranks(jnp.concatenate([o, s]))
    loss = pl.pallas_call(
        _combine_kernel,
        out_shape=jax.ShapeDtypeStruct((1, 1), jnp.float32),
    )(o.reshape(1, N), s.reshape(1, N), ranks)
    return loss.reshape(())
